# baseline (speedup 1.0000x reference)
import functools

import jax
import jax.numpy as jnp
from jax import lax
from jax.experimental import pallas as pl
from jax.experimental.pallas import tpu as pltpu
from jax.experimental.pallas import tpu_sc as plsc

N, E, D, G, HEADS = 10000, 320000, 128, 64, 8
NP = 10240

NC = 2
NS = 16
EPS = E // NS
EPW = E // (NS * NC)
C = 80
RPT = NP // NS

_mesh_cache = []


def _mesh():
    if not _mesh_cache:
        _mesh_cache.append(plsc.VectorSubcoreMesh(
            core_axis_name="c", subcore_axis_name="s"))
    return _mesh_cache[0]


def _zero_fill(buf, n16):
    z = jnp.zeros((16,), jnp.float32)

    def body(i, _):
        buf[pl.ds(i * 16, 16)] = z
        return 0

    lax.fori_loop(0, n16, body, 0)


def _zero_fill2d(buf, nrows, ncols):
    z = jnp.zeros((16,), jnp.float32)

    def body(r, _):
        for k in range(ncols // 16):
            buf[r, pl.ds(k * 16, 16)] = z
        return 0

    lax.fori_loop(0, nrows, body, 0)


@functools.partial(jax.jit, static_argnames=("H",))
def _edge_scalars(asd, src2, dst2, *, H):
    HPC = max(H // NC, 1)
    SEG = 125
    NCH = EPS // C if H > 1 else EPW // C
    NSEG = NCH // SEG

    @functools.partial(
        pl.kernel, mesh=_mesh(),
        compiler_params=pltpu.CompilerParams(use_tc_tiling_on_sc=False, needs_layout_passes=False),
        out_type=(jax.ShapeDtypeStruct((H, E), jnp.float32),
                  jax.ShapeDtypeStruct((NC * HPC, NP), jnp.float32)),
        scratch_types=[
            pltpu.VMEM((HPC, 2, NP), jnp.float32),
            pltpu.VMEM((SEG, C), jnp.int32),
            pltpu.VMEM((SEG, C), jnp.int32),
            pltpu.VMEM((2, HPC, C), jnp.float32),
            pltpu.VMEM((RPT,), jnp.float32),
            pltpu.VMEM_SHARED((HPC, NP), jnp.float32),
            pltpu.SemaphoreType.DMA,
            pltpu.SemaphoreType.DMA,
            pltpu.SemaphoreType.DMA,
            pltpu.SemaphoreType.DMA,
        ])
    def kern(asd_h, src_h, dst_h, ee_h, den_h, tabs, src_l, dst_l, eebuf,
             zbuf, dsh, st0, st1, sc0, sc1):
        c = lax.axis_index("c")
        s = lax.axis_index("s")
        sst = (st0, st1)
        ssc = (sc0, sc1)
        if H > 1:
            row0 = s * NCH
        else:
            row0 = (s * NC + c) * NCH

        _zero_fill(zbuf, RPT // 16)
        for hh in range(HPC):
            pltpu.sync_copy(zbuf, dsh.at[hh, pl.ds(s * RPT, RPT)])
        plsc.subcore_barrier()

        for hh in range(HPC):
            if H > 1:
                pltpu.sync_copy(asd_h.at[0, c * HPC + hh], tabs.at[hh, 0])
                pltpu.sync_copy(asd_h.at[1, c * HPC + hh], tabs.at[hh, 1])
            else:
                pltpu.sync_copy(asd_h.at[0, 0], tabs.at[hh, 0])
                pltpu.sync_copy(asd_h.at[1, 0], tabs.at[hh, 1])

        dummy = ee_h.at[0, pl.ds(0, C)]

        def compute(ci, b):
            for hh in range(HPC):
                for j in range(C // 16):
                    s16 = src_l[ci, pl.ds(j * 16, 16)]
                    d16 = dst_l[ci, pl.ds(j * 16, 16)]
                    av = plsc.load_gather(tabs.at[hh, 0], [s16])
                    bv = plsc.load_gather(tabs.at[hh, 1], [d16])
                    e16 = av + bv
                    e16 = jnp.where(e16 >= 0.0, e16, e16 * 0.2)
                    eebuf[b, hh, pl.ds(j * 16, 16)] = jnp.exp(e16)

        def issue(g, ci, b):
            gbase = (row0 + g * SEG + ci) * C
            for hh in range(HPC):
                hrow = (c * HPC + hh) if H > 1 else 0
                pltpu.async_copy(eebuf.at[b, hh],
                                 ee_h.at[hrow, pl.ds(gbase, C)], sst[b])
                pltpu.async_copy(eebuf.at[b, hh], dsh.at[hh].at[dst_l.at[ci]],
                                 ssc[b], add=True)

        def drain(b):
            for hh in range(HPC):
                pltpu.make_async_copy(dummy, eebuf.at[b, hh], sst[b]).wait()
                pltpu.make_async_copy(dummy, eebuf.at[b, hh], ssc[b]).wait()

        def seg_body(g, _):
            pltpu.sync_copy(src_h.at[pl.ds(row0 + g * SEG, SEG)], src_l)
            pltpu.sync_copy(dst_h.at[pl.ds(row0 + g * SEG, SEG)], dst_l)

            def pair(i, _):
                c0 = 2 * i
                c1 = c0 + 1

                @pl.when(i > 0)
                def _():
                    drain(0)
                compute(c0, 0)
                issue(g, c0, 0)

                @pl.when(i > 0)
                def _():
                    drain(1)
                compute(c1, 1)
                issue(g, c1, 1)
                return 0

            lax.fori_loop(0, SEG // 2, pair, 0)
            if SEG % 2 == 1:
                drain(0)
                compute(SEG - 1, 0)
                issue(g, SEG - 1, 0)
            drain(0)
            drain(1)
            return 0

        lax.fori_loop(0, NSEG, seg_body, 0)
        plsc.subcore_barrier()

        for hh in range(HPC):
            pltpu.sync_copy(dsh.at[hh, pl.ds(s * RPT, RPT)],
                            den_h.at[c * HPC + hh, pl.ds(s * RPT, RPT)])

    return kern(asd, src2, dst2)


@functools.partial(jax.jit, static_argnames=("slots", "weighted"))
def _spmm(table, src2, dst2, ee2=None, *, slots, weighted):
    NCH = (EPS if slots == 2 else EPW) // C
    SEG = 50 if slots == 2 else 25
    NSEG = NCH // SEG

    @functools.partial(
        pl.kernel, mesh=_mesh(),
        compiler_params=pltpu.CompilerParams(use_tc_tiling_on_sc=False, needs_layout_passes=False),
        out_type=jax.ShapeDtypeStruct((NC, NP, D), jnp.float32),
        scratch_types=[
            pltpu.VMEM((SEG, C), jnp.int32),
            pltpu.VMEM((SEG, C), jnp.int32),
            pltpu.VMEM((SEG, C), jnp.float32),
            pltpu.VMEM((2, C, D), jnp.float32),
            pltpu.VMEM_SHARED((NP, D), jnp.float32),
            pltpu.SemaphoreType.DMA,
            pltpu.SemaphoreType.DMA,
            pltpu.SemaphoreType.DMA,
            pltpu.SemaphoreType.DMA,
        ])
    def kern(tbl_h, src_h, dst_h, ee_h, out_h, src_l, dst_l, ee_l, rows,
             acc, sg0, sg1, ss0, ss1):
        c = lax.axis_index("c")
        s = lax.axis_index("s")
        sg = (sg0, sg1)
        ss = (ss0, ss1)
        if slots == 2:
            row0 = s * NCH
        else:
            row0 = (s * NC + c) * NCH

        _zero_fill2d(rows.at[0], C, D)
        for r in range(RPT // C):
            pltpu.sync_copy(rows.at[0], acc.at[pl.ds(s * RPT + r * C, C)])
        plsc.subcore_barrier()

        tbl = tbl_h.at[c] if slots == 2 else tbl_h.at[0]
        dummy = tbl_h.at[0, pl.ds(0, C)]

        def g_issue(ci, b):
            pltpu.async_copy(tbl.at[src_l.at[ci]], rows.at[b], sg[b])

        def g_wait(b):
            pltpu.make_async_copy(dummy, rows.at[b], sg[b]).wait()

        def s_issue(ci, b):
            pltpu.async_copy(rows.at[b], acc.at[dst_l.at[ci]], ss[b],
                             add=True)

        def s_wait(b):
            pltpu.make_async_copy(dummy, rows.at[b], ss[b]).wait()

        def scale(ci, b):
            if not weighted:
                return
            rb = rows.at[b]

            def jbody(jj, _):
                w16 = ee_l[ci, pl.ds(jj * 16, 16)]
                for l in range(16):
                    wr = w16[l]
                    for k in range(D // 16):
                        sl = pl.ds(k * 16, 16)
                        rb[jj * 16 + l, sl] = rb[jj * 16 + l, sl] * wr
                return 0

            lax.fori_loop(0, C // 16, jbody, 0)

        def seg_body(g, _):
            pltpu.sync_copy(src_h.at[pl.ds(row0 + g * SEG, SEG)], src_l)
            pltpu.sync_copy(dst_h.at[pl.ds(row0 + g * SEG, SEG)], dst_l)
            if weighted:
                if slots == 2:
                    pltpu.sync_copy(ee_h.at[c].at[pl.ds(row0 + g * SEG, SEG)],
                                    ee_l)
                else:
                    pltpu.sync_copy(ee_h.at[0].at[pl.ds(row0 + g * SEG, SEG)],
                                    ee_l)
            g_issue(0, 0)

            def pair(i, _):
                c0 = 2 * i
                c1 = c0 + 1

                @pl.when(i > 0)
                def _():
                    s_wait(1)
                g_issue(c1, 1)
                g_wait(0)
                scale(c0, 0)
                s_issue(c0, 0)
                g_wait(1)
                scale(c1, 1)
                s_wait(0)
                @pl.when(c0 + 2 < SEG)
                def _():
                    g_issue(c0 + 2, 0)
                s_issue(c1, 1)
                return 0

            lax.fori_loop(0, SEG // 2, pair, 0)
            if SEG % 2 == 1:
                g_wait(0)
                scale(SEG - 1, 0)
                s_issue(SEG - 1, 0)
                s_wait(0)
            s_wait(1)
            return 0

        lax.fori_loop(0, NSEG, seg_body, 0)

        plsc.subcore_barrier()
        pltpu.sync_copy(acc.at[pl.ds(s * RPT, RPT)],
                        out_h.at[c].at[pl.ds(s * RPT, RPT)])

    if ee2 is None:
        ee2 = jnp.zeros((slots, NCH, C), jnp.float32)
    return kern(table, src2, dst2, ee2)


@jax.jit
def _degree(dst):
    @functools.partial(
        pl.kernel, mesh=_mesh(),
        compiler_params=pltpu.CompilerParams(use_tc_tiling_on_sc=False, needs_layout_passes=False),
        out_type=jax.ShapeDtypeStruct((NC, NP), jnp.float32),
        scratch_types=[
            pltpu.VMEM((C,), jnp.int32),
            pltpu.VMEM((C,), jnp.float32),
            pltpu.VMEM((RPT,), jnp.float32),
            pltpu.VMEM_SHARED((NP,), jnp.float32),
            pltpu.SemaphoreType.DMA,
        ])
    def kern(dst_h, out_h, dbuf, ones, zbuf, acc, sem):
        c = lax.axis_index("c")
        s = lax.axis_index("s")
        _zero_fill(zbuf, RPT // 16)
        pltpu.sync_copy(zbuf, acc.at[pl.ds(s * RPT, RPT)])
        o = jnp.ones((16,), jnp.float32)
        for j in range(C // 16):
            ones[pl.ds(j * 16, 16)] = o
        plsc.subcore_barrier()

        def chunk(i, _):
            base = (s * NC + c) * EPW + i * C
            pltpu.sync_copy(dst_h.at[pl.ds(base, C)], dbuf)
            pltpu.sync_copy(ones, acc.at[dbuf], add=True)
            return 0

        lax.fori_loop(0, EPW // C, chunk, 0)
        plsc.subcore_barrier()
        pltpu.sync_copy(acc.at[pl.ds(s * RPT, RPT)],
                        out_h.at[c].at[pl.ds(s * RPT, RPT)])

    return kern(dst)


@functools.partial(jax.jit, static_argnames=("act",))
def _mm(x, w, b=None, *, act="none"):
    n, k = x.shape
    m = w.shape[1]
    br = 512 if n % 512 == 0 else n
    has_b = b is not None

    def body(x_ref, w_ref, b_ref, o_ref):
        y = jnp.dot(x_ref[...], w_ref[...],
                    preferred_element_type=jnp.float32,
                    precision=jax.lax.Precision.HIGHEST)
        if has_b:
            y = y + b_ref[...]
        if act == "relu":
            y = jnp.maximum(y, 0.0)
        o_ref[...] = y

    b2 = (b if has_b else jnp.zeros((m,), jnp.float32)).reshape(1, m)
    return pl.pallas_call(
        body,
        grid=(n // br,),
        in_specs=[
            pl.BlockSpec((br, k), lambda i: (i, 0)),
            pl.BlockSpec((k, m), lambda i: (0, 0)),
            pl.BlockSpec((1, m), lambda i: (0, 0)),
        ],
        out_specs=pl.BlockSpec((br, m), lambda i: (i, 0)),
        out_shape=jax.ShapeDtypeStruct((n, m), jnp.float32),
    )(x, w, b2)


def _asd(hflat, att_s, att_d):
    H = att_s.shape[0]
    eye = jnp.eye(H, dtype=jnp.float32)
    As = att_s[:, :, None] * eye[:, None, :]
    Ad = att_d[:, :, None] * eye[:, None, :]
    A = jnp.concatenate([As, Ad], axis=2).reshape(H * D, 2 * H)
    A = jnp.pad(A, ((0, 0), (0, 128 - 2 * H)))
    out = _mm(hflat, A)
    return out[:, :2 * H].T.reshape(2, H, NP)


def _gat1_hop(h, att_s, att_d, src, dst, src2, dst2):
    hflat = h.transpose(1, 0, 2).reshape(NP, HEADS * D)
    asd = _asd(hflat, att_s, att_d)
    ee, den = _edge_scalars(asd, src2, dst2, H=HEADS)
    rden = 1.0 / (den + 1e-16)
    ee2 = ee.reshape(HEADS, E // C, C)
    outs = [_spmm(h[2 * k:2 * k + 2], src2, dst2, ee2[2 * k:2 * k + 2],
                  slots=2, weighted=True) for k in range(HEADS // 2)]
    out = jnp.concatenate(outs, axis=0)
    return out * rden[:, :, None]


def _gat2_hops(g, p, src, dst, src2, dst2, hops):
    h = _mm(g, p['gat2_W'])
    for _ in range(hops):
        asd = _asd(h, p['gat2_as'], p['gat2_ad'])
        ee, den = _edge_scalars(asd, src2, dst2, H=1)
        rden = 1.0 / (den[0] + den[1] + 1e-16)
        part = _spmm(h[None], src2, dst2, ee.reshape(1, E // C, C),
                     slots=1, weighted=True)
        h = (part[0] + part[1]) * rden[:, None]
    h = jax.nn.relu(h + p['gat2_b'])
    h = _mm(h, p['gatA_W'], p['gatA_b'], act="relu")
    h = _mm(h, p['gatB_W'], p['gatB_b'], act="relu")
    return _mm(h, p['gatC_W'], p['gatC_b'])


def _gcn_layer(h_in, src2, dst2, W, b, dinv, inv_deg, hops):
    h = _mm(h_in, W)
    nslab = h.shape[1] // D
    for _ in range(hops):
        hs = h * dinv[:, None]
        slabs = hs.reshape(NP, nslab, D).transpose(1, 0, 2)
        outs = []
        for k in range(nslab // 2):
            o = _spmm(slabs[2 * k:2 * k + 2], src2, dst2, slots=2,
                      weighted=False)
            outs.append(o)
        agg = jnp.concatenate(outs, axis=0).transpose(1, 0, 2).reshape(NP, -1)
        h = agg * dinv[:, None] + h * inv_deg[:, None]
    return h + b


def kernel(x, edge_index, batch, params):
    p = params
    src = edge_index[0]
    dst = edge_index[1]
    src2 = src.reshape(E // C, C)
    dst2 = dst.reshape(E // C, C)
    xp = jnp.pad(x, ((0, NP - N), (0, 0)))

    h = _mm(xp, p['gat1_W']).reshape(NP, HEADS, D).transpose(1, 0, 2)
    g = []
    for _ in range(3):
        h = _gat1_hop(h, p['gat1_as'], p['gat1_ad'], src, dst, src2, dst2)
        g.append(jax.nn.elu(
            h.transpose(1, 0, 2).reshape(NP, HEADS * D) + p['gat1_b']))

    b1 = jax.nn.relu(_gat2_hops(g[0], p, src, dst, src2, dst2, 1))
    b2 = jax.nn.relu(_gat2_hops(g[1], p, src, dst, src2, dst2, 2))
    h3 = jax.nn.relu(_gat2_hops(g[2], p, src, dst, src2, dst2, 3))

    degp = _degree(dst)
    deg = 1.0 + degp[0] + degp[1]
    dinv = deg ** -0.5
    inv_deg = dinv * dinv

    h1 = jax.nn.relu(_gcn_layer(b1, src2, dst2, p['gcn2_W'], p['gcn2_b'],
                                dinv, inv_deg, 1))
    h1 = jax.nn.relu(_gcn_layer(h1, src2, dst2, p['gcn3_W'], p['gcn3_b'],
                                dinv, inv_deg, 1))
    h2 = jax.nn.relu(_gcn_layer(b2, src2, dst2, p['gcn2_W'], p['gcn2_b'],
                                dinv, inv_deg, 2))

    a = _mm(h1, p['hwA_W'], p['hwA_b'])
    b = _mm(h2, p['hwB_W'], p['hwB_b'])
    z = jax.nn.sigmoid(a + b)
    hmix = z * b + (1.0 - z) * a
    gi = _mm(h3, p['gru_Wi'], p['gru_bi'])
    gh = _mm(hmix, p['gru_Wh'], p['gru_bh'])
    i_r, i_z, i_n = jnp.split(gi, 3, axis=-1)
    h_r, h_z, h_n = jnp.split(gh, 3, axis=-1)
    r = jax.nn.sigmoid(i_r + h_r)
    zz = jax.nn.sigmoid(i_z + h_z)
    nn_ = jnp.tanh(i_n + r * h_n)
    concat = (1.0 - zz) * nn_ + zz * hmix

    vi = jax.ops.segment_max(concat[:N], batch, num_segments=G)
    vi = jnp.where(jnp.isfinite(vi), vi, 0.0)
    vi = _mm(vi, p['fA_W'], p['fA_b'], act="relu")
    fBW = jnp.pad(p['fB_W'], ((0, 0), (0, 8)))
    fBb = jnp.pad(p['fB_b'], (0, 8))
    return _mm(vi, fBW, fBb)[:, :120]


# speedup vs baseline: 1.0126x; 1.0126x over previous
import functools

import jax
import jax.numpy as jnp
from jax import lax
from jax.experimental import pallas as pl
from jax.experimental.pallas import tpu as pltpu
from jax.experimental.pallas import tpu_sc as plsc

N, E, D, G, HEADS = 10000, 320000, 128, 64, 8
NP = 10240

NC = 2
NS = 16
EPS = E // NS
EPW = E // (NS * NC)
C = 80
RPT = NP // NS

_mesh_cache = []


def _mesh():
    if not _mesh_cache:
        _mesh_cache.append(plsc.VectorSubcoreMesh(
            core_axis_name="c", subcore_axis_name="s"))
    return _mesh_cache[0]


def _zero_fill(buf, n16):
    z = jnp.zeros((16,), jnp.float32)

    def body(i, _):
        buf[pl.ds(i * 16, 16)] = z
        return 0

    lax.fori_loop(0, n16, body, 0)


def _zero_fill2d(buf, nrows, ncols):
    z = jnp.zeros((16,), jnp.float32)

    def body(r, _):
        for k in range(ncols // 16):
            buf[r, pl.ds(k * 16, 16)] = z
        return 0

    lax.fori_loop(0, nrows, body, 0)


@functools.partial(jax.jit, static_argnames=("H",))
def _edge_scalars(asd, src2, dst2, *, H):
    HPC = max(H // NC, 1)
    SEG = 125
    NCH = EPS // C if H > 1 else EPW // C
    NSEG = NCH // SEG

    @functools.partial(
        pl.kernel, mesh=_mesh(),
        compiler_params=pltpu.CompilerParams(use_tc_tiling_on_sc=False, needs_layout_passes=False),
        out_type=(jax.ShapeDtypeStruct((H, E), jnp.float32),
                  jax.ShapeDtypeStruct((NC * HPC, NP), jnp.float32)),
        scratch_types=[
            pltpu.VMEM((HPC, 2, NP), jnp.float32),
            pltpu.VMEM((SEG, C), jnp.int32),
            pltpu.VMEM((SEG, C), jnp.int32),
            pltpu.VMEM((2, HPC, C), jnp.float32),
            pltpu.VMEM((RPT,), jnp.float32),
            pltpu.VMEM_SHARED((HPC, NP), jnp.float32),
            pltpu.SemaphoreType.DMA,
            pltpu.SemaphoreType.DMA,
            pltpu.SemaphoreType.DMA,
            pltpu.SemaphoreType.DMA,
        ])
    def kern(asd_h, src_h, dst_h, ee_h, den_h, tabs, src_l, dst_l, eebuf,
             zbuf, dsh, st0, st1, sc0, sc1):
        c = lax.axis_index("c")
        s = lax.axis_index("s")
        sst = (st0, st1)
        ssc = (sc0, sc1)
        if H > 1:
            row0 = s * NCH
        else:
            row0 = (s * NC + c) * NCH

        _zero_fill(zbuf, RPT // 16)
        for hh in range(HPC):
            pltpu.sync_copy(zbuf, dsh.at[hh, pl.ds(s * RPT, RPT)])
        plsc.subcore_barrier()

        for hh in range(HPC):
            if H > 1:
                pltpu.sync_copy(asd_h.at[0, c * HPC + hh], tabs.at[hh, 0])
                pltpu.sync_copy(asd_h.at[1, c * HPC + hh], tabs.at[hh, 1])
            else:
                pltpu.sync_copy(asd_h.at[0, 0], tabs.at[hh, 0])
                pltpu.sync_copy(asd_h.at[1, 0], tabs.at[hh, 1])

        dummy = ee_h.at[0, pl.ds(0, C)]

        def compute(ci, b):
            for hh in range(HPC):
                for j in range(C // 16):
                    s16 = src_l[ci, pl.ds(j * 16, 16)]
                    d16 = dst_l[ci, pl.ds(j * 16, 16)]
                    av = plsc.load_gather(tabs.at[hh, 0], [s16])
                    bv = plsc.load_gather(tabs.at[hh, 1], [d16])
                    e16 = av + bv
                    e16 = jnp.where(e16 >= 0.0, e16, e16 * 0.2)
                    eebuf[b, hh, pl.ds(j * 16, 16)] = jnp.exp(e16)

        def issue(g, ci, b):
            gbase = (row0 + g * SEG + ci) * C
            for hh in range(HPC):
                hrow = (c * HPC + hh) if H > 1 else 0
                pltpu.async_copy(eebuf.at[b, hh],
                                 ee_h.at[hrow, pl.ds(gbase, C)], sst[b])
                pltpu.async_copy(eebuf.at[b, hh], dsh.at[hh].at[dst_l.at[ci]],
                                 ssc[b], add=True)

        def drain(b):
            for hh in range(HPC):
                pltpu.make_async_copy(dummy, eebuf.at[b, hh], sst[b]).wait()
                pltpu.make_async_copy(dummy, eebuf.at[b, hh], ssc[b]).wait()

        def seg_body(g, _):
            pltpu.sync_copy(src_h.at[pl.ds(row0 + g * SEG, SEG)], src_l)
            pltpu.sync_copy(dst_h.at[pl.ds(row0 + g * SEG, SEG)], dst_l)

            def pair(i, _):
                c0 = 2 * i
                c1 = c0 + 1

                @pl.when(i > 0)
                def _():
                    drain(0)
                compute(c0, 0)
                issue(g, c0, 0)

                @pl.when(i > 0)
                def _():
                    drain(1)
                compute(c1, 1)
                issue(g, c1, 1)
                return 0

            lax.fori_loop(0, SEG // 2, pair, 0)
            if SEG % 2 == 1:
                drain(0)
                compute(SEG - 1, 0)
                issue(g, SEG - 1, 0)
            drain(0)
            drain(1)
            return 0

        lax.fori_loop(0, NSEG, seg_body, 0)
        plsc.subcore_barrier()

        for hh in range(HPC):
            pltpu.sync_copy(dsh.at[hh, pl.ds(s * RPT, RPT)],
                            den_h.at[c * HPC + hh, pl.ds(s * RPT, RPT)])

    return kern(asd, src2, dst2)


@functools.partial(jax.jit, static_argnames=("slots", "weighted"))
def _spmm(table, src2, dst2, ee2=None, *, slots, weighted):
    NCH = (EPS if slots == 2 else EPW) // C
    SEG = 50 if slots == 2 else 25
    NSEG = NCH // SEG

    @functools.partial(
        pl.kernel, mesh=_mesh(),
        compiler_params=pltpu.CompilerParams(use_tc_tiling_on_sc=False, needs_layout_passes=False),
        out_type=jax.ShapeDtypeStruct((NC, NP, D), jnp.float32),
        scratch_types=[
            pltpu.VMEM((SEG, C), jnp.int32),
            pltpu.VMEM((SEG, C), jnp.int32),
            pltpu.VMEM((SEG, C), jnp.float32),
            pltpu.VMEM((2, C, D), jnp.float32),
            pltpu.VMEM_SHARED((NP, D), jnp.float32),
            pltpu.SemaphoreType.DMA,
            pltpu.SemaphoreType.DMA,
            pltpu.SemaphoreType.DMA,
            pltpu.SemaphoreType.DMA,
        ])
    def kern(tbl_h, src_h, dst_h, ee_h, out_h, src_l, dst_l, ee_l, rows,
             acc, sg0, sg1, ss0, ss1):
        c = lax.axis_index("c")
        s = lax.axis_index("s")
        sg = (sg0, sg1)
        ss = (ss0, ss1)
        if slots == 2:
            row0 = s * NCH
        else:
            row0 = (s * NC + c) * NCH

        _zero_fill2d(rows.at[0], C, D)
        for r in range(RPT // C):
            pltpu.sync_copy(rows.at[0], acc.at[pl.ds(s * RPT + r * C, C)])
        plsc.subcore_barrier()

        tbl = tbl_h.at[c] if slots == 2 else tbl_h.at[0]
        dummy = tbl_h.at[0, pl.ds(0, C)]

        def g_issue(ci, b):
            pltpu.async_copy(tbl.at[src_l.at[ci]], rows.at[b], sg[b])

        def g_wait(b):
            pltpu.make_async_copy(dummy, rows.at[b], sg[b]).wait()

        def s_issue(ci, b):
            pltpu.async_copy(rows.at[b], acc.at[dst_l.at[ci]], ss[b],
                             add=True)

        def s_wait(b):
            pltpu.make_async_copy(dummy, rows.at[b], ss[b]).wait()

        def scale(ci, b):
            if not weighted:
                return
            rb = rows.at[b]

            def jbody(jj, _):
                w16 = ee_l[ci, pl.ds(jj * 16, 16)]
                for l in range(16):
                    wr = w16[l]
                    for k in range(D // 16):
                        sl = pl.ds(k * 16, 16)
                        rb[jj * 16 + l, sl] = rb[jj * 16 + l, sl] * wr
                return 0

            lax.fori_loop(0, C // 16, jbody, 0)

        def seg_body(g, _):
            pltpu.sync_copy(src_h.at[pl.ds(row0 + g * SEG, SEG)], src_l)
            pltpu.sync_copy(dst_h.at[pl.ds(row0 + g * SEG, SEG)], dst_l)
            if weighted:
                if slots == 2:
                    pltpu.sync_copy(ee_h.at[c].at[pl.ds(row0 + g * SEG, SEG)],
                                    ee_l)
                else:
                    pltpu.sync_copy(ee_h.at[0].at[pl.ds(row0 + g * SEG, SEG)],
                                    ee_l)
            g_issue(0, 0)

            def pair(i, _):
                c0 = 2 * i
                c1 = c0 + 1

                @pl.when(i > 0)
                def _():
                    s_wait(1)
                g_issue(c1, 1)
                g_wait(0)
                scale(c0, 0)
                s_issue(c0, 0)
                g_wait(1)
                scale(c1, 1)
                s_wait(0)
                @pl.when(c0 + 2 < SEG)
                def _():
                    g_issue(c0 + 2, 0)
                s_issue(c1, 1)
                return 0

            lax.fori_loop(0, SEG // 2, pair, 0)
            if SEG % 2 == 1:
                g_wait(0)
                scale(SEG - 1, 0)
                s_issue(SEG - 1, 0)
                s_wait(0)
            s_wait(1)
            return 0

        lax.fori_loop(0, NSEG, seg_body, 0)

        plsc.subcore_barrier()
        pltpu.sync_copy(acc.at[pl.ds(s * RPT, RPT)],
                        out_h.at[c].at[pl.ds(s * RPT, RPT)])

    if ee2 is None:
        ee2 = jnp.zeros((slots, NCH, C), jnp.float32)
    return kern(table, src2, dst2, ee2)


@jax.jit
def _degree(dst):
    @functools.partial(
        pl.kernel, mesh=_mesh(),
        compiler_params=pltpu.CompilerParams(use_tc_tiling_on_sc=False, needs_layout_passes=False),
        out_type=jax.ShapeDtypeStruct((NC, NP), jnp.float32),
        scratch_types=[
            pltpu.VMEM((C,), jnp.int32),
            pltpu.VMEM((C,), jnp.float32),
            pltpu.VMEM((RPT,), jnp.float32),
            pltpu.VMEM_SHARED((NP,), jnp.float32),
            pltpu.SemaphoreType.DMA,
        ])
    def kern(dst_h, out_h, dbuf, ones, zbuf, acc, sem):
        c = lax.axis_index("c")
        s = lax.axis_index("s")
        _zero_fill(zbuf, RPT // 16)
        pltpu.sync_copy(zbuf, acc.at[pl.ds(s * RPT, RPT)])
        o = jnp.ones((16,), jnp.float32)
        for j in range(C // 16):
            ones[pl.ds(j * 16, 16)] = o
        plsc.subcore_barrier()

        def chunk(i, _):
            base = (s * NC + c) * EPW + i * C
            pltpu.sync_copy(dst_h.at[pl.ds(base, C)], dbuf)
            pltpu.sync_copy(ones, acc.at[dbuf], add=True)
            return 0

        lax.fori_loop(0, EPW // C, chunk, 0)
        plsc.subcore_barrier()
        pltpu.sync_copy(acc.at[pl.ds(s * RPT, RPT)],
                        out_h.at[c].at[pl.ds(s * RPT, RPT)])

    return kern(dst)


@functools.partial(jax.jit, static_argnames=("act",))
def _mm(x, w, b=None, *, act="none"):
    n, k = x.shape
    m = w.shape[1]
    br = 512 if n % 512 == 0 else n
    has_b = b is not None

    def body(x_ref, w_ref, b_ref, o_ref):
        y = jnp.dot(x_ref[...], w_ref[...],
                    preferred_element_type=jnp.float32,
                    precision=jax.lax.Precision.HIGHEST)
        if has_b:
            y = y + b_ref[...]
        if act == "relu":
            y = jnp.maximum(y, 0.0)
        o_ref[...] = y

    b2 = (b if has_b else jnp.zeros((m,), jnp.float32)).reshape(1, m)
    return pl.pallas_call(
        body,
        grid=(n // br,),
        in_specs=[
            pl.BlockSpec((br, k), lambda i: (i, 0)),
            pl.BlockSpec((k, m), lambda i: (0, 0)),
            pl.BlockSpec((1, m), lambda i: (0, 0)),
        ],
        out_specs=pl.BlockSpec((br, m), lambda i: (i, 0)),
        out_shape=jax.ShapeDtypeStruct((n, m), jnp.float32),
    )(x, w, b2)


def _asd(hflat, att_s, att_d):
    H = att_s.shape[0]
    eye = jnp.eye(H, dtype=jnp.float32)
    As = att_s[:, :, None] * eye[:, None, :]
    Ad = att_d[:, :, None] * eye[:, None, :]
    A = jnp.concatenate([As, Ad], axis=2).reshape(H * D, 2 * H)
    A = jnp.pad(A, ((0, 0), (0, 128 - 2 * H)))
    out = _mm(hflat, A)
    return out[:, :2 * H].T.reshape(2, H, NP)


@jax.jit
def _segmax(xc, starts):
    GB = 8

    def body(st_ref, x_ref, o_ref):
        g0 = pl.program_id(0) * GB
        for gg in range(GB):
            lo = st_ref[g0 + gg]
            hi = st_ref[g0 + gg + 1]

            def rbody(i, acc):
                idx = i * 8 + jax.lax.broadcasted_iota(jnp.int32, (8, D), 0)
                blk = x_ref[pl.ds(i * 8, 8), :]
                m = (idx >= lo) & (idx < hi)
                return jnp.maximum(acc, jnp.max(
                    jnp.where(m, blk, -jnp.inf), axis=0))

            acc0 = jnp.full((D,), -jnp.inf, jnp.float32)
            acc = lax.fori_loop(lo // 8, (hi + 7) // 8, rbody, acc0)
            o_ref[gg, :] = jnp.where(acc > -3e38, acc, 0.0)

    return pl.pallas_call(
        body,
        grid=(G // GB,),
        in_specs=[
            pl.BlockSpec(memory_space=pltpu.SMEM),
            pl.BlockSpec((NP, D), lambda i: (0, 0)),
        ],
        out_specs=pl.BlockSpec((GB, D), lambda i: (i, 0)),
        out_shape=jax.ShapeDtypeStruct((G, D), jnp.float32),
    )(starts, xc)


def _gat1_hop(h, att_s, att_d, src, dst, src2, dst2):
    hflat = h.transpose(1, 0, 2).reshape(NP, HEADS * D)
    asd = _asd(hflat, att_s, att_d)
    ee, den = _edge_scalars(asd, src2, dst2, H=HEADS)
    rden = 1.0 / (den + 1e-16)
    ee2 = ee.reshape(HEADS, E // C, C)
    outs = [_spmm(h[2 * k:2 * k + 2], src2, dst2, ee2[2 * k:2 * k + 2],
                  slots=2, weighted=True) for k in range(HEADS // 2)]
    out = jnp.concatenate(outs, axis=0)
    return out * rden[:, :, None]


def _gat2_hops(g, p, src, dst, src2, dst2, hops):
    h = _mm(g, p['gat2_W'])
    for _ in range(hops):
        asd = _asd(h, p['gat2_as'], p['gat2_ad'])
        ee, den = _edge_scalars(asd, src2, dst2, H=1)
        rden = 1.0 / (den[0] + den[1] + 1e-16)
        part = _spmm(h[None], src2, dst2, ee.reshape(1, E // C, C),
                     slots=1, weighted=True)
        h = (part[0] + part[1]) * rden[:, None]
    h = jax.nn.relu(h + p['gat2_b'])
    h = _mm(h, p['gatA_W'], p['gatA_b'], act="relu")
    h = _mm(h, p['gatB_W'], p['gatB_b'], act="relu")
    return _mm(h, p['gatC_W'], p['gatC_b'])


def _gcn_layer(h_in, src2, dst2, W, b, dinv, inv_deg, hops):
    h = _mm(h_in, W)
    nslab = h.shape[1] // D
    for _ in range(hops):
        hs = h * dinv[:, None]
        slabs = hs.reshape(NP, nslab, D).transpose(1, 0, 2)
        outs = []
        for k in range(nslab // 2):
            o = _spmm(slabs[2 * k:2 * k + 2], src2, dst2, slots=2,
                      weighted=False)
            outs.append(o)
        agg = jnp.concatenate(outs, axis=0).transpose(1, 0, 2).reshape(NP, -1)
        h = agg * dinv[:, None] + h * inv_deg[:, None]
    return h + b


def kernel(x, edge_index, batch, params):
    p = params
    src = edge_index[0]
    dst = edge_index[1]
    src2 = src.reshape(E // C, C)
    dst2 = dst.reshape(E // C, C)
    xp = jnp.pad(x, ((0, NP - N), (0, 0)))

    h = _mm(xp, p['gat1_W']).reshape(NP, HEADS, D).transpose(1, 0, 2)
    g = []
    for _ in range(3):
        h = _gat1_hop(h, p['gat1_as'], p['gat1_ad'], src, dst, src2, dst2)
        g.append(jax.nn.elu(
            h.transpose(1, 0, 2).reshape(NP, HEADS * D) + p['gat1_b']))

    b1 = jax.nn.relu(_gat2_hops(g[0], p, src, dst, src2, dst2, 1))
    b2 = jax.nn.relu(_gat2_hops(g[1], p, src, dst, src2, dst2, 2))
    h3 = jax.nn.relu(_gat2_hops(g[2], p, src, dst, src2, dst2, 3))

    degp = _degree(dst)
    deg = 1.0 + degp[0] + degp[1]
    dinv = deg ** -0.5
    inv_deg = dinv * dinv

    h1 = jax.nn.relu(_gcn_layer(b1, src2, dst2, p['gcn2_W'], p['gcn2_b'],
                                dinv, inv_deg, 1))
    h1 = jax.nn.relu(_gcn_layer(h1, src2, dst2, p['gcn3_W'], p['gcn3_b'],
                                dinv, inv_deg, 1))
    h2 = jax.nn.relu(_gcn_layer(b2, src2, dst2, p['gcn2_W'], p['gcn2_b'],
                                dinv, inv_deg, 2))

    a = _mm(h1, p['hwA_W'], p['hwA_b'])
    b = _mm(h2, p['hwB_W'], p['hwB_b'])
    z = jax.nn.sigmoid(a + b)
    hmix = z * b + (1.0 - z) * a
    gi = _mm(h3, p['gru_Wi'], p['gru_bi'])
    gh = _mm(hmix, p['gru_Wh'], p['gru_bh'])
    i_r, i_z, i_n = jnp.split(gi, 3, axis=-1)
    h_r, h_z, h_n = jnp.split(gh, 3, axis=-1)
    r = jax.nn.sigmoid(i_r + h_r)
    zz = jax.nn.sigmoid(i_z + h_z)
    nn_ = jnp.tanh(i_n + r * h_n)
    concat = (1.0 - zz) * nn_ + zz * hmix

    starts = jnp.searchsorted(batch, jnp.arange(G + 1, dtype=jnp.int32)
                              ).astype(jnp.int32)
    vi = _segmax(concat, starts)
    vi = _mm(vi, p['fA_W'], p['fA_b'], act="relu")
    fBW = jnp.pad(p['fB_W'], ((0, 0), (0, 8)))
    fBb = jnp.pad(p['fB_b'], (0, 8))
    return _mm(vi, fBW, fBb)[:, :120]
